# baseline (speedup 1.0000x reference)
import functools

import jax
import jax.numpy as jnp
from jax import lax
from jax.experimental import pallas as pl
from jax.experimental.pallas import tpu as pltpu
from jax.experimental.pallas import tpu_sc as plsc

_NC = 2
_NS = 16
_NW = _NC * _NS
_EB = 128
_NPH = 2


def _cdiv(a, b):
    return -(-a // b)


@functools.cache
def _build_deg(acc_rows, nblk, d):
    rpt = acc_rows // _NS
    mesh = plsc.VectorSubcoreMesh(core_axis_name="c", subcore_axis_name="s")

    @functools.partial(
        pl.kernel, mesh=mesh,
        out_type=jax.ShapeDtypeStruct((_NC, acc_rows, d), jnp.float32),
        scratch_types=[
            pltpu.VMEM_SHARED((acc_rows, d), jnp.float32),
            pltpu.VMEM((nblk, _EB), jnp.int32),
            pltpu.VMEM((_EB, d), jnp.float32),
        ],
    )
    def deg_kernel(row_hbm, z_hbm, ones_hbm, out_hbm, acc, rowb, onesv):
        c = lax.axis_index("c")
        s = lax.axis_index("s")
        wid = c * _NS + s
        pltpu.sync_copy(row_hbm.at[wid], rowb)
        pltpu.sync_copy(z_hbm, acc.at[pl.ds(s * rpt, rpt)])
        pltpu.sync_copy(ones_hbm, onesv)
        plsc.subcore_barrier()

        def step(i, carry):
            pltpu.sync_copy(onesv, acc.at[rowb.at[i]], add=True)
            return carry

        lax.fori_loop(0, nblk, step, 0)
        plsc.subcore_barrier()
        pltpu.sync_copy(acc.at[pl.ds(s * rpt, rpt)],
                        out_hbm.at[c, pl.ds(s * rpt, rpt)])

    return deg_kernel


@functools.cache
def _build_spmm(n, d, acc_rows, nbp):
    rpt = acc_rows // _NS
    mesh = plsc.VectorSubcoreMesh(core_axis_name="c", subcore_axis_name="s")

    @functools.partial(
        pl.kernel, mesh=mesh,
        out_type=jax.ShapeDtypeStruct((_NC, acc_rows, d), jnp.float32),
        scratch_types=[
            pltpu.VMEM_SHARED((acc_rows, d), jnp.float32),
            pltpu.VMEM((nbp, _EB), jnp.int32),
            pltpu.VMEM((nbp, _EB), jnp.int32),
            pltpu.VMEM((_EB, d), jnp.float32),
            pltpu.VMEM((_EB, d), jnp.float32),
            pltpu.SemaphoreType.DMA,
            pltpu.SemaphoreType.DMA,
        ],
    )
    def spmm_kernel(hs_hbm, col_hbm, row_hbm, z_hbm, out_hbm,
                    acc, colb, rowb, gbuf0, gbuf1, sem0, sem1):
        c = lax.axis_index("c")
        s = lax.axis_index("s")
        wid = c * _NS + s
        pltpu.sync_copy(z_hbm, acc.at[pl.ds(s * rpt, rpt)])
        plsc.subcore_barrier()

        for p in range(_NPH):
            def step(i, carry, p=p):
                pltpu.sync_copy(col_hbm.at[wid, p, i], colb.at[0])
                pltpu.sync_copy(row_hbm.at[wid, p, i], rowb.at[0])
                pltpu.async_copy(hs_hbm.at[colb.at[0]], gbuf0, sem0).wait()
                pltpu.sync_copy(gbuf0, acc.at[rowb.at[0]], add=True)
                return carry

            lax.fori_loop(0, nbp, step, 0)

        plsc.subcore_barrier()
        pltpu.sync_copy(acc.at[pl.ds(s * rpt, rpt)],
                        out_hbm.at[c, pl.ds(s * rpt, rpt)])

    return spmm_kernel


@functools.cache
def _build_mlp(n, d, hid, bn):
    ng = n // bn

    def body(x_ref, wm_ref, bm_ref, wf_ref, bf_ref, sdeg_ref,
             h_ref, hs_ref, h0_ref, qs_ref, dis_ref, dinv_ref, qacc):
        i = pl.program_id(0)
        deg = (sdeg_ref[0, :, 0] + sdeg_ref[1, :, 0] + 1.0)[:, None]
        dis = lax.rsqrt(deg)
        dinv = 1.0 / deg
        dis_ref[...] = dis
        dinv_ref[...] = dinv
        h = jnp.maximum(
            jnp.dot(x_ref[...], wm_ref[...],
                    preferred_element_type=jnp.float32) + bm_ref[...], 0.0)
        h_ref[...] = h
        hs_ref[...] = dis * h
        h0 = jnp.dot(h, wf_ref[...],
                     preferred_element_type=jnp.float32) + bf_ref[...]
        h0_ref[...] = h0
        colsum = jnp.sum(h0, axis=0, keepdims=True)

        @pl.when(i == 0)
        def _():
            qacc[...] = colsum

        @pl.when(i > 0)
        def _():
            qacc[...] = qacc[...] + colsum

        qs_ref[...] = qacc[...]

    return pl.pallas_call(
        body,
        grid=(ng,),
        in_specs=[
            pl.BlockSpec((bn, d), lambda i: (i, 0)),
            pl.BlockSpec((d, hid), lambda i: (0, 0)),
            pl.BlockSpec((1, hid), lambda i: (0, 0)),
            pl.BlockSpec((hid, hid), lambda i: (0, 0)),
            pl.BlockSpec((1, hid), lambda i: (0, 0)),
            pl.BlockSpec((_NC, bn, d), lambda i: (0, i, 0)),
        ],
        out_specs=[
            pl.BlockSpec((bn, hid), lambda i: (i, 0)),
            pl.BlockSpec((bn, hid), lambda i: (i, 0)),
            pl.BlockSpec((bn, hid), lambda i: (i, 0)),
            pl.BlockSpec((1, hid), lambda i: (0, 0)),
            pl.BlockSpec((bn, 1), lambda i: (i, 0)),
            pl.BlockSpec((bn, 1), lambda i: (i, 0)),
        ],
        out_shape=[
            jax.ShapeDtypeStruct((n, hid), jnp.float32),
            jax.ShapeDtypeStruct((n, hid), jnp.float32),
            jax.ShapeDtypeStruct((n, hid), jnp.float32),
            jax.ShapeDtypeStruct((1, hid), jnp.float32),
            jax.ShapeDtypeStruct((n, 1), jnp.float32),
            jax.ShapeDtypeStruct((n, 1), jnp.float32),
        ],
        scratch_shapes=[pltpu.VMEM((1, hid), jnp.float32)],
    )


@functools.cache
def _build_rec(n, hid, bn):
    ng = n // bn

    def body(coef_ref, s_ref, zl_ref, zp_ref, dis_ref, dinv_ref,
             wf_ref, bf_ref, zn_ref, hs_ref, hk_ref, qs_ref, qacc):
        i = pl.program_id(0)
        c0 = coef_ref[0]
        c1 = coef_ref[1]
        c2 = coef_ref[2]
        spart = s_ref[0] + s_ref[1]
        dis = dis_ref[...]
        zl = zl_ref[...]
        z = c0 * (dis * spart + dinv_ref[...] * zl) + c1 * zl + c2 * zp_ref[...]
        zn_ref[...] = z
        hs_ref[...] = dis * z
        hk = jnp.dot(z, wf_ref[...],
                     preferred_element_type=jnp.float32) + bf_ref[...]
        hk_ref[...] = hk
        colsum = jnp.sum(hk, axis=0, keepdims=True)

        @pl.when(i == 0)
        def _():
            qacc[...] = colsum

        @pl.when(i > 0)
        def _():
            qacc[...] = qacc[...] + colsum

        qs_ref[...] = qacc[...]

    return pl.pallas_call(
        body,
        grid=(ng,),
        in_specs=[
            pl.BlockSpec(memory_space=pltpu.SMEM),
            pl.BlockSpec((_NC, bn, hid), lambda i: (0, i, 0)),
            pl.BlockSpec((bn, hid), lambda i: (i, 0)),
            pl.BlockSpec((bn, hid), lambda i: (i, 0)),
            pl.BlockSpec((bn, 1), lambda i: (i, 0)),
            pl.BlockSpec((bn, 1), lambda i: (i, 0)),
            pl.BlockSpec((hid, hid), lambda i: (0, 0)),
            pl.BlockSpec((1, hid), lambda i: (0, 0)),
        ],
        out_specs=[
            pl.BlockSpec((bn, hid), lambda i: (i, 0)),
            pl.BlockSpec((bn, hid), lambda i: (i, 0)),
            pl.BlockSpec((bn, hid), lambda i: (i, 0)),
            pl.BlockSpec((1, hid), lambda i: (0, 0)),
        ],
        out_shape=[
            jax.ShapeDtypeStruct((n, hid), jnp.float32),
            jax.ShapeDtypeStruct((n, hid), jnp.float32),
            jax.ShapeDtypeStruct((n, hid), jnp.float32),
            jax.ShapeDtypeStruct((1, hid), jnp.float32),
        ],
        scratch_shapes=[pltpu.VMEM((1, hid), jnp.float32)],
    )


@functools.cache
def _build_attn(n, hid, out_dim, kp1, bn):
    ng = n // bn
    inv_n = 1.0 / n

    def body(*refs):
        h_refs = refs[:kp1]
        qs_ref, wc_ref, bc_ref = refs[kp1:kp1 + 3]
        out_ref, zt_ref = refs[kp1 + 3:kp1 + 5]
        q = qs_ref[...] * inv_n
        scores = []
        hs = []
        for k in range(kp1):
            hk = h_refs[k][...]
            hs.append(hk)
            scores.append(jnp.sum(hk * q[k][None, :], axis=1, keepdims=True))
        sc = jnp.tanh(jnp.concatenate(scores, axis=1))
        m = jnp.max(sc, axis=1, keepdims=True)
        e = jnp.exp(sc - m)
        alpha = e / jnp.sum(e, axis=1, keepdims=True)
        zt = alpha[:, 0:1] * hs[0]
        for k in range(1, kp1):
            zt = zt + alpha[:, k:k + 1] * hs[k]
        zt = jnp.maximum(zt, 0.0)
        zt_ref[...] = zt
        out_ref[...] = jnp.dot(zt, wc_ref[...],
                               preferred_element_type=jnp.float32) + bc_ref[...]

    in_specs = [pl.BlockSpec((bn, hid), lambda i: (i, 0)) for _ in range(kp1)]
    in_specs += [
        pl.BlockSpec((kp1, hid), lambda i: (0, 0)),
        pl.BlockSpec((hid, out_dim), lambda i: (0, 0)),
        pl.BlockSpec((1, out_dim), lambda i: (0, 0)),
    ]
    return pl.pallas_call(
        body,
        grid=(ng,),
        in_specs=in_specs,
        out_specs=[
            pl.BlockSpec((bn, out_dim), lambda i: (i, 0)),
            pl.BlockSpec((bn, hid), lambda i: (i, 0)),
        ],
        out_shape=[
            jax.ShapeDtypeStruct((n, out_dim), jnp.float32),
            jax.ShapeDtypeStruct((n, hid), jnp.float32),
        ],
    )


def kernel(x, edge_index, W_mlp, b_mlp, W_filters, b_filters, W_cls, b_cls):
    n, d = x.shape
    hid = W_mlp.shape[1]
    out_dim = W_cls.shape[1]
    kp1 = W_filters.shape[0]
    e = edge_index.shape[1]

    nbp = 2 * _cdiv(_cdiv(e, _NW * _EB), 2 * _NPH)
    nblk = _NPH * nbp
    e_pad = nblk * _NW * _EB
    acc_rows = _cdiv(n + 1, _NS * 8) * _NS * 8
    rpt = acc_rows // _NS
    bn = 1000 if n % 1000 == 0 else (500 if n % 500 == 0 else 8)

    row = edge_index[0]
    col = edge_index[1]
    npad = e_pad - e
    if npad:
        row = jnp.concatenate([row, jnp.full((npad,), n, jnp.int32)])
        col = jnp.concatenate([col, jnp.zeros((npad,), jnp.int32)])
    row_f = row.reshape(_NW, nblk, _EB)
    row = row.reshape(_NW, _NPH, nbp, _EB)
    col = col.reshape(_NW, _NPH, nbp, _EB)

    z_d = jnp.zeros((rpt, d), jnp.float32)
    ones_d = jnp.ones((_EB, d), jnp.float32)

    sdeg = _build_deg(acc_rows, nblk, d)(row_f, z_d, ones_d)

    h, hs, h0, qs0, dis, dinv = _build_mlp(n, d, hid, bn)(
        x, W_mlp, b_mlp.reshape(1, hid), W_filters[0],
        b_filters[0].reshape(1, hid), sdeg)

    a_c, b_c = 1.0, 1.0
    spmm = _build_spmm(n, d, acc_rows, nbp)
    rec = _build_rec(n, hid, bn)

    hs_list = [h0]
    qs_list = [qs0]
    zl, zp = h, h
    hs_cur = hs
    for k_idx in range(1, kp1):
        s = spmm(hs_cur, col, row, z_d)
        if k_idx == 1:
            c0 = (a_c + b_c + 2.0) / 2.0
            c1 = (a_c - b_c) / 2.0
            c2 = 0.0
        else:
            k = float(k_idx)
            c0 = (2 * k + a_c + b_c) * (2 * k + a_c + b_c - 1) / (
                2 * k * (k + a_c + b_c))
            c1 = (2 * k + a_c + b_c - 1) * (a_c ** 2 - b_c ** 2) / (
                2 * k * (k + a_c + b_c) * (2 * k + a_c + b_c - 2))
            c2 = -((k + a_c - 1) * (k + b_c - 1) * (2 * k + a_c + b_c) / (
                k * (k + a_c + b_c) * (2 * k + a_c + b_c - 2)))
        coefs = jnp.array([c0, c1, c2], jnp.float32)
        zn, hsn, hk, qsk = rec(coefs, s, zl, zp, dis, dinv,
                               W_filters[k_idx], b_filters[k_idx].reshape(1, hid))
        zp, zl, hs_cur = zl, zn, hsn
        hs_list.append(hk)
        qs_list.append(qsk)

    qsum = jnp.concatenate(qs_list, axis=0)
    out, zt = _build_attn(n, hid, out_dim, kp1, bn)(
        *hs_list, qsum, W_cls, b_cls.reshape(1, out_dim))
    return out, zt


# speedup vs baseline: 1.5012x; 1.5012x over previous
import functools

import jax
import jax.numpy as jnp
from jax import lax
from jax.experimental import pallas as pl
from jax.experimental.pallas import tpu as pltpu
from jax.experimental.pallas import tpu_sc as plsc

_NC = 2
_NS = 16
_NW = _NC * _NS
_EB = 128


def _cdiv(a, b):
    return -(-a // b)


@functools.cache
def _build_deg(acc_rows, e_pad):
    ept = e_pad // _NW
    nblk = ept // _EB
    rpt = acc_rows // _NS
    mesh = plsc.VectorSubcoreMesh(core_axis_name="c", subcore_axis_name="s")

    @functools.partial(
        pl.kernel, mesh=mesh,
        out_type=jax.ShapeDtypeStruct((_NC, acc_rows, 16), jnp.float32),
        scratch_types=[
            pltpu.VMEM_SHARED((acc_rows, 16), jnp.float32),
            pltpu.VMEM((_EB,), jnp.int32),
            pltpu.VMEM((_EB, 16), jnp.float32),
        ],
    )
    def deg_kernel(row_hbm, z_hbm, ones_hbm, out_hbm, acc, rowv, onesv):
        c = lax.axis_index("c")
        s = lax.axis_index("s")
        wid = c * _NS + s
        pltpu.sync_copy(z_hbm, acc.at[pl.ds(s * rpt, rpt)])
        pltpu.sync_copy(ones_hbm, onesv)
        plsc.subcore_barrier()
        base = wid * ept

        def step(i, carry):
            pltpu.sync_copy(row_hbm.at[pl.ds(base + i * _EB, _EB)], rowv)
            pltpu.sync_copy(onesv, acc.at[rowv], add=True)
            return carry

        lax.fori_loop(0, nblk, step, 0)
        plsc.subcore_barrier()
        pltpu.sync_copy(acc.at[pl.ds(s * rpt, rpt)],
                        out_hbm.at[c, pl.ds(s * rpt, rpt)])

    return deg_kernel


@functools.cache
def _build_spmm(n, d, acc_rows, e_pad):
    ept = e_pad // _NW
    nblk = ept // _EB
    rpt = acc_rows // _NS
    mesh = plsc.VectorSubcoreMesh(core_axis_name="c", subcore_axis_name="s")

    @functools.partial(
        pl.kernel, mesh=mesh,
        out_type=jax.ShapeDtypeStruct((_NC, acc_rows, d), jnp.float32),
        scratch_types=[
            pltpu.VMEM_SHARED((acc_rows, d), jnp.float32),
            pltpu.VMEM((_EB,), jnp.int32),
            pltpu.VMEM((_EB,), jnp.int32),
            pltpu.VMEM((_EB, d), jnp.float32),
            pltpu.SemaphoreType.DMA,
        ],
    )
    def spmm_kernel(hs_hbm, col_hbm, row_hbm, z_hbm, out_hbm,
                    acc, colv, rowv, gbuf, sem):
        c = lax.axis_index("c")
        s = lax.axis_index("s")
        wid = c * _NS + s
        pltpu.sync_copy(z_hbm, acc.at[pl.ds(s * rpt, rpt)])
        plsc.subcore_barrier()
        base = wid * ept

        def step(i, carry):
            off = base + i * _EB
            pltpu.sync_copy(col_hbm.at[pl.ds(off, _EB)], colv)
            pltpu.sync_copy(row_hbm.at[pl.ds(off, _EB)], rowv)
            pltpu.async_copy(hs_hbm.at[colv], gbuf, sem).wait()
            pltpu.sync_copy(gbuf, acc.at[rowv], add=True)
            return carry

        lax.fori_loop(0, nblk, step, 0)
        plsc.subcore_barrier()
        pltpu.sync_copy(acc.at[pl.ds(s * rpt, rpt)],
                        out_hbm.at[c, pl.ds(s * rpt, rpt)])

    return spmm_kernel


@functools.cache
def _build_mlp(n, d, hid, bn):
    ng = n // bn

    def body(x_ref, wm_ref, bm_ref, wf_ref, bf_ref, sdeg_ref,
             h_ref, hs_ref, h0_ref, qs_ref, dis_ref, dinv_ref, qacc):
        i = pl.program_id(0)
        deg = (sdeg_ref[0, :, 0] + sdeg_ref[1, :, 0] + 1.0)[:, None]
        dis = lax.rsqrt(deg)
        dinv = 1.0 / deg
        dis_ref[...] = dis
        dinv_ref[...] = dinv
        h = jnp.maximum(
            jnp.dot(x_ref[...], wm_ref[...],
                    preferred_element_type=jnp.float32) + bm_ref[...], 0.0)
        h_ref[...] = h
        hs_ref[...] = dis * h
        h0 = jnp.dot(h, wf_ref[...],
                     preferred_element_type=jnp.float32) + bf_ref[...]
        h0_ref[...] = h0
        colsum = jnp.sum(h0, axis=0, keepdims=True)

        @pl.when(i == 0)
        def _():
            qacc[...] = colsum

        @pl.when(i > 0)
        def _():
            qacc[...] = qacc[...] + colsum

        qs_ref[...] = qacc[...]

    return pl.pallas_call(
        body,
        grid=(ng,),
        in_specs=[
            pl.BlockSpec((bn, d), lambda i: (i, 0)),
            pl.BlockSpec((d, hid), lambda i: (0, 0)),
            pl.BlockSpec((1, hid), lambda i: (0, 0)),
            pl.BlockSpec((hid, hid), lambda i: (0, 0)),
            pl.BlockSpec((1, hid), lambda i: (0, 0)),
            pl.BlockSpec((_NC, bn, 16), lambda i: (0, i, 0)),
        ],
        out_specs=[
            pl.BlockSpec((bn, hid), lambda i: (i, 0)),
            pl.BlockSpec((bn, hid), lambda i: (i, 0)),
            pl.BlockSpec((bn, hid), lambda i: (i, 0)),
            pl.BlockSpec((1, hid), lambda i: (0, 0)),
            pl.BlockSpec((bn, 1), lambda i: (i, 0)),
            pl.BlockSpec((bn, 1), lambda i: (i, 0)),
        ],
        out_shape=[
            jax.ShapeDtypeStruct((n, hid), jnp.float32),
            jax.ShapeDtypeStruct((n, hid), jnp.float32),
            jax.ShapeDtypeStruct((n, hid), jnp.float32),
            jax.ShapeDtypeStruct((1, hid), jnp.float32),
            jax.ShapeDtypeStruct((n, 1), jnp.float32),
            jax.ShapeDtypeStruct((n, 1), jnp.float32),
        ],
        scratch_shapes=[pltpu.VMEM((1, hid), jnp.float32)],
    )


@functools.cache
def _build_rec(n, hid, bn):
    ng = n // bn

    def body(coef_ref, s_ref, zl_ref, zp_ref, dis_ref, dinv_ref,
             wf_ref, bf_ref, zn_ref, hs_ref, hk_ref, qs_ref, qacc):
        i = pl.program_id(0)
        c0 = coef_ref[0]
        c1 = coef_ref[1]
        c2 = coef_ref[2]
        spart = s_ref[0] + s_ref[1]
        dis = dis_ref[...]
        zl = zl_ref[...]
        z = c0 * (dis * spart + dinv_ref[...] * zl) + c1 * zl + c2 * zp_ref[...]
        zn_ref[...] = z
        hs_ref[...] = dis * z
        hk = jnp.dot(z, wf_ref[...],
                     preferred_element_type=jnp.float32) + bf_ref[...]
        hk_ref[...] = hk
        colsum = jnp.sum(hk, axis=0, keepdims=True)

        @pl.when(i == 0)
        def _():
            qacc[...] = colsum

        @pl.when(i > 0)
        def _():
            qacc[...] = qacc[...] + colsum

        qs_ref[...] = qacc[...]

    return pl.pallas_call(
        body,
        grid=(ng,),
        in_specs=[
            pl.BlockSpec(memory_space=pltpu.SMEM),
            pl.BlockSpec((_NC, bn, hid), lambda i: (0, i, 0)),
            pl.BlockSpec((bn, hid), lambda i: (i, 0)),
            pl.BlockSpec((bn, hid), lambda i: (i, 0)),
            pl.BlockSpec((bn, 1), lambda i: (i, 0)),
            pl.BlockSpec((bn, 1), lambda i: (i, 0)),
            pl.BlockSpec((hid, hid), lambda i: (0, 0)),
            pl.BlockSpec((1, hid), lambda i: (0, 0)),
        ],
        out_specs=[
            pl.BlockSpec((bn, hid), lambda i: (i, 0)),
            pl.BlockSpec((bn, hid), lambda i: (i, 0)),
            pl.BlockSpec((bn, hid), lambda i: (i, 0)),
            pl.BlockSpec((1, hid), lambda i: (0, 0)),
        ],
        out_shape=[
            jax.ShapeDtypeStruct((n, hid), jnp.float32),
            jax.ShapeDtypeStruct((n, hid), jnp.float32),
            jax.ShapeDtypeStruct((n, hid), jnp.float32),
            jax.ShapeDtypeStruct((1, hid), jnp.float32),
        ],
        scratch_shapes=[pltpu.VMEM((1, hid), jnp.float32)],
    )


@functools.cache
def _build_attn(n, hid, out_dim, kp1, bn):
    ng = n // bn
    inv_n = 1.0 / n

    def body(*refs):
        h_refs = refs[:kp1]
        qs_ref, wc_ref, bc_ref = refs[kp1:kp1 + 3]
        out_ref, zt_ref = refs[kp1 + 3:kp1 + 5]
        q = qs_ref[...] * inv_n
        scores = []
        hs = []
        for k in range(kp1):
            hk = h_refs[k][...]
            hs.append(hk)
            scores.append(jnp.sum(hk * q[k][None, :], axis=1, keepdims=True))
        sc = jnp.tanh(jnp.concatenate(scores, axis=1))
        m = jnp.max(sc, axis=1, keepdims=True)
        e = jnp.exp(sc - m)
        alpha = e / jnp.sum(e, axis=1, keepdims=True)
        zt = alpha[:, 0:1] * hs[0]
        for k in range(1, kp1):
            zt = zt + alpha[:, k:k + 1] * hs[k]
        zt = jnp.maximum(zt, 0.0)
        zt_ref[...] = zt
        out_ref[...] = jnp.dot(zt, wc_ref[...],
                               preferred_element_type=jnp.float32) + bc_ref[...]

    in_specs = [pl.BlockSpec((bn, hid), lambda i: (i, 0)) for _ in range(kp1)]
    in_specs += [
        pl.BlockSpec((kp1, hid), lambda i: (0, 0)),
        pl.BlockSpec((hid, out_dim), lambda i: (0, 0)),
        pl.BlockSpec((1, out_dim), lambda i: (0, 0)),
    ]
    return pl.pallas_call(
        body,
        grid=(ng,),
        in_specs=in_specs,
        out_specs=[
            pl.BlockSpec((bn, out_dim), lambda i: (i, 0)),
            pl.BlockSpec((bn, hid), lambda i: (i, 0)),
        ],
        out_shape=[
            jax.ShapeDtypeStruct((n, out_dim), jnp.float32),
            jax.ShapeDtypeStruct((n, hid), jnp.float32),
        ],
    )


def kernel(x, edge_index, W_mlp, b_mlp, W_filters, b_filters, W_cls, b_cls):
    n, d = x.shape
    hid = W_mlp.shape[1]
    out_dim = W_cls.shape[1]
    kp1 = W_filters.shape[0]
    e = edge_index.shape[1]

    e_pad = _cdiv(e, _NW * _EB) * _NW * _EB
    acc_rows = _cdiv(n + 1, _NS * 8) * _NS * 8
    rpt = acc_rows // _NS
    bn = 1000 if n % 1000 == 0 else (500 if n % 500 == 0 else 8)

    row = edge_index[0]
    col = edge_index[1]
    npad = e_pad - e
    if npad:
        row = jnp.concatenate([row, jnp.full((npad,), n, jnp.int32)])
        col = jnp.concatenate([col, jnp.zeros((npad,), jnp.int32)])

    z_d = jnp.zeros((rpt, d), jnp.float32)
    z_16 = jnp.zeros((rpt, 16), jnp.float32)
    ones_16 = jnp.ones((_EB, 16), jnp.float32)

    sdeg = _build_deg(acc_rows, e_pad)(row, z_16, ones_16)

    h, hs, h0, qs0, dis, dinv = _build_mlp(n, d, hid, bn)(
        x, W_mlp, b_mlp.reshape(1, hid), W_filters[0],
        b_filters[0].reshape(1, hid), sdeg)

    a_c, b_c = 1.0, 1.0
    spmm = _build_spmm(n, d, acc_rows, e_pad)
    rec = _build_rec(n, hid, bn)

    hs_list = [h0]
    qs_list = [qs0]
    zl, zp = h, h
    hs_cur = hs
    for k_idx in range(1, kp1):
        s = spmm(hs_cur, col, row, z_d)
        if k_idx == 1:
            c0 = (a_c + b_c + 2.0) / 2.0
            c1 = (a_c - b_c) / 2.0
            c2 = 0.0
        else:
            k = float(k_idx)
            c0 = (2 * k + a_c + b_c) * (2 * k + a_c + b_c - 1) / (
                2 * k * (k + a_c + b_c))
            c1 = (2 * k + a_c + b_c - 1) * (a_c ** 2 - b_c ** 2) / (
                2 * k * (k + a_c + b_c) * (2 * k + a_c + b_c - 2))
            c2 = -((k + a_c - 1) * (k + b_c - 1) * (2 * k + a_c + b_c) / (
                k * (k + a_c + b_c) * (2 * k + a_c + b_c - 2)))
        coefs = jnp.array([c0, c1, c2], jnp.float32)
        zn, hsn, hk, qsk = rec(coefs, s, zl, zp, dis, dinv,
                               W_filters[k_idx], b_filters[k_idx].reshape(1, hid))
        zp, zl, hs_cur = zl, zn, hsn
        hs_list.append(hk)
        qs_list.append(qsk)

    qsum = jnp.concatenate(qs_list, axis=0)
    out, zt = _build_attn(n, hid, out_dim, kp1, bn)(
        *hs_list, qsum, W_cls, b_cls.reshape(1, out_dim))
    return out, zt


# speedup vs baseline: 1.7182x; 1.1446x over previous
import functools

import jax
import jax.numpy as jnp
from jax import lax
from jax.experimental import pallas as pl
from jax.experimental.pallas import tpu as pltpu
from jax.experimental.pallas import tpu_sc as plsc

_NC = 2
_NS = 16
_NW = _NC * _NS
_EB = 128


def _cdiv(a, b):
    return -(-a // b)


@functools.cache
def _build_deg(acc_rows, e_pad):
    ept = e_pad // _NW
    nblk = ept // _EB
    rpt = acc_rows // _NS
    mesh = plsc.VectorSubcoreMesh(core_axis_name="c", subcore_axis_name="s")

    @functools.partial(
        pl.kernel, mesh=mesh,
        out_type=jax.ShapeDtypeStruct((_NC, acc_rows, 16), jnp.float32),
        scratch_types=[
            pltpu.VMEM_SHARED((acc_rows, 16), jnp.float32),
            pltpu.VMEM((_EB,), jnp.int32),
            pltpu.VMEM((_EB, 16), jnp.float32),
        ],
    )
    def deg_kernel(row_hbm, z_hbm, ones_hbm, out_hbm, acc, rowv, onesv):
        c = lax.axis_index("c")
        s = lax.axis_index("s")
        wid = c * _NS + s
        pltpu.sync_copy(z_hbm, acc.at[pl.ds(s * rpt, rpt)])
        pltpu.sync_copy(ones_hbm, onesv)
        plsc.subcore_barrier()
        base = wid * ept

        def step(i, carry):
            pltpu.sync_copy(row_hbm.at[pl.ds(base + i * _EB, _EB)], rowv)
            pltpu.sync_copy(onesv, acc.at[rowv], add=True)
            return carry

        lax.fori_loop(0, nblk, step, 0)
        plsc.subcore_barrier()
        pltpu.sync_copy(acc.at[pl.ds(s * rpt, rpt)],
                        out_hbm.at[c, pl.ds(s * rpt, rpt)])

    return deg_kernel


_NPH = 2


@functools.cache
def _build_spmm(n, d, acc_rows, nbp):
    rpt = acc_rows // _NS
    mesh = plsc.VectorSubcoreMesh(core_axis_name="c", subcore_axis_name="s")

    @functools.partial(
        pl.kernel, mesh=mesh,
        out_type=jax.ShapeDtypeStruct((_NC, acc_rows, d), jnp.float32),
        scratch_types=[
            pltpu.VMEM_SHARED((acc_rows, d), jnp.float32),
            pltpu.VMEM((nbp, _EB), jnp.int32),
            pltpu.VMEM((nbp, _EB), jnp.int32),
            pltpu.VMEM((_EB, d), jnp.float32),
            pltpu.VMEM((_EB, d), jnp.float32),
            pltpu.SemaphoreType.DMA,
            pltpu.SemaphoreType.DMA,
        ],
    )
    def spmm_kernel(hs_hbm, col_hbm, row_hbm, z_hbm, out_hbm,
                    acc, colb, rowb, gbuf0, gbuf1, sem0, sem1):
        c = lax.axis_index("c")
        s = lax.axis_index("s")
        wid = c * _NS + s
        pltpu.sync_copy(z_hbm, acc.at[pl.ds(s * rpt, rpt)])
        plsc.subcore_barrier()

        def step(j, carry):
            i0 = 2 * j
            pltpu.async_copy(hs_hbm.at[colb.at[i0 + 1]], gbuf1, sem1)
            pltpu.make_async_copy(hs_hbm.at[colb.at[i0]], gbuf0, sem0).wait()
            pltpu.sync_copy(gbuf0, acc.at[rowb.at[i0]], add=True)

            @pl.when(i0 + 2 < nbp)
            def _():
                pltpu.async_copy(hs_hbm.at[colb.at[i0 + 2]], gbuf0, sem0)

            pltpu.make_async_copy(hs_hbm.at[colb.at[i0 + 1]], gbuf1, sem1).wait()
            pltpu.sync_copy(gbuf1, acc.at[rowb.at[i0 + 1]], add=True)
            return carry

        for p in range(_NPH):
            pltpu.sync_copy(col_hbm.at[wid, p], colb)
            pltpu.sync_copy(row_hbm.at[wid, p], rowb)
            pltpu.async_copy(hs_hbm.at[colb.at[0]], gbuf0, sem0)
            lax.fori_loop(0, nbp // 2, step, 0)

        plsc.subcore_barrier()
        pltpu.sync_copy(acc.at[pl.ds(s * rpt, rpt)],
                        out_hbm.at[c, pl.ds(s * rpt, rpt)])

    return spmm_kernel


@functools.cache
def _build_mlp(n, d, hid, bn):
    ng = n // bn

    def body(x_ref, wm_ref, bm_ref, wf_ref, bf_ref, sdeg_ref,
             h_ref, hs_ref, h0_ref, qs_ref, dis_ref, dinv_ref, qacc):
        i = pl.program_id(0)
        deg = (sdeg_ref[0, :, 0] + sdeg_ref[1, :, 0] + 1.0)[:, None]
        dis = lax.rsqrt(deg)
        dinv = 1.0 / deg
        dis_ref[...] = dis
        dinv_ref[...] = dinv
        h = jnp.maximum(
            jnp.dot(x_ref[...], wm_ref[...],
                    preferred_element_type=jnp.float32) + bm_ref[...], 0.0)
        h_ref[...] = h
        hs_ref[...] = dis * h
        h0 = jnp.dot(h, wf_ref[...],
                     preferred_element_type=jnp.float32) + bf_ref[...]
        h0_ref[...] = h0
        colsum = jnp.sum(h0, axis=0, keepdims=True)

        @pl.when(i == 0)
        def _():
            qacc[...] = colsum

        @pl.when(i > 0)
        def _():
            qacc[...] = qacc[...] + colsum

        qs_ref[...] = qacc[...]

    return pl.pallas_call(
        body,
        grid=(ng,),
        in_specs=[
            pl.BlockSpec((bn, d), lambda i: (i, 0)),
            pl.BlockSpec((d, hid), lambda i: (0, 0)),
            pl.BlockSpec((1, hid), lambda i: (0, 0)),
            pl.BlockSpec((hid, hid), lambda i: (0, 0)),
            pl.BlockSpec((1, hid), lambda i: (0, 0)),
            pl.BlockSpec((_NC, bn, 16), lambda i: (0, i, 0)),
        ],
        out_specs=[
            pl.BlockSpec((bn, hid), lambda i: (i, 0)),
            pl.BlockSpec((bn, hid), lambda i: (i, 0)),
            pl.BlockSpec((bn, hid), lambda i: (i, 0)),
            pl.BlockSpec((1, hid), lambda i: (0, 0)),
            pl.BlockSpec((bn, 1), lambda i: (i, 0)),
            pl.BlockSpec((bn, 1), lambda i: (i, 0)),
        ],
        out_shape=[
            jax.ShapeDtypeStruct((n, hid), jnp.float32),
            jax.ShapeDtypeStruct((n, hid), jnp.float32),
            jax.ShapeDtypeStruct((n, hid), jnp.float32),
            jax.ShapeDtypeStruct((1, hid), jnp.float32),
            jax.ShapeDtypeStruct((n, 1), jnp.float32),
            jax.ShapeDtypeStruct((n, 1), jnp.float32),
        ],
        scratch_shapes=[pltpu.VMEM((1, hid), jnp.float32)],
    )


@functools.cache
def _build_rec(n, hid, bn):
    ng = n // bn

    def body(coef_ref, s_ref, zl_ref, zp_ref, dis_ref, dinv_ref,
             wf_ref, bf_ref, zn_ref, hs_ref, hk_ref, qs_ref, qacc):
        i = pl.program_id(0)
        c0 = coef_ref[0]
        c1 = coef_ref[1]
        c2 = coef_ref[2]
        spart = s_ref[0] + s_ref[1]
        dis = dis_ref[...]
        zl = zl_ref[...]
        z = c0 * (dis * spart + dinv_ref[...] * zl) + c1 * zl + c2 * zp_ref[...]
        zn_ref[...] = z
        hs_ref[...] = dis * z
        hk = jnp.dot(z, wf_ref[...],
                     preferred_element_type=jnp.float32) + bf_ref[...]
        hk_ref[...] = hk
        colsum = jnp.sum(hk, axis=0, keepdims=True)

        @pl.when(i == 0)
        def _():
            qacc[...] = colsum

        @pl.when(i > 0)
        def _():
            qacc[...] = qacc[...] + colsum

        qs_ref[...] = qacc[...]

    return pl.pallas_call(
        body,
        grid=(ng,),
        in_specs=[
            pl.BlockSpec(memory_space=pltpu.SMEM),
            pl.BlockSpec((_NC, bn, hid), lambda i: (0, i, 0)),
            pl.BlockSpec((bn, hid), lambda i: (i, 0)),
            pl.BlockSpec((bn, hid), lambda i: (i, 0)),
            pl.BlockSpec((bn, 1), lambda i: (i, 0)),
            pl.BlockSpec((bn, 1), lambda i: (i, 0)),
            pl.BlockSpec((hid, hid), lambda i: (0, 0)),
            pl.BlockSpec((1, hid), lambda i: (0, 0)),
        ],
        out_specs=[
            pl.BlockSpec((bn, hid), lambda i: (i, 0)),
            pl.BlockSpec((bn, hid), lambda i: (i, 0)),
            pl.BlockSpec((bn, hid), lambda i: (i, 0)),
            pl.BlockSpec((1, hid), lambda i: (0, 0)),
        ],
        out_shape=[
            jax.ShapeDtypeStruct((n, hid), jnp.float32),
            jax.ShapeDtypeStruct((n, hid), jnp.float32),
            jax.ShapeDtypeStruct((n, hid), jnp.float32),
            jax.ShapeDtypeStruct((1, hid), jnp.float32),
        ],
        scratch_shapes=[pltpu.VMEM((1, hid), jnp.float32)],
    )


@functools.cache
def _build_attn(n, hid, out_dim, kp1, bn):
    ng = n // bn
    inv_n = 1.0 / n

    def body(*refs):
        h_refs = refs[:kp1]
        qs_ref, wc_ref, bc_ref = refs[kp1:kp1 + 3]
        out_ref, zt_ref = refs[kp1 + 3:kp1 + 5]
        q = qs_ref[...] * inv_n
        scores = []
        hs = []
        for k in range(kp1):
            hk = h_refs[k][...]
            hs.append(hk)
            scores.append(jnp.sum(hk * q[k][None, :], axis=1, keepdims=True))
        sc = jnp.tanh(jnp.concatenate(scores, axis=1))
        m = jnp.max(sc, axis=1, keepdims=True)
        e = jnp.exp(sc - m)
        alpha = e / jnp.sum(e, axis=1, keepdims=True)
        zt = alpha[:, 0:1] * hs[0]
        for k in range(1, kp1):
            zt = zt + alpha[:, k:k + 1] * hs[k]
        zt = jnp.maximum(zt, 0.0)
        zt_ref[...] = zt
        out_ref[...] = jnp.dot(zt, wc_ref[...],
                               preferred_element_type=jnp.float32) + bc_ref[...]

    in_specs = [pl.BlockSpec((bn, hid), lambda i: (i, 0)) for _ in range(kp1)]
    in_specs += [
        pl.BlockSpec((kp1, hid), lambda i: (0, 0)),
        pl.BlockSpec((hid, out_dim), lambda i: (0, 0)),
        pl.BlockSpec((1, out_dim), lambda i: (0, 0)),
    ]
    return pl.pallas_call(
        body,
        grid=(ng,),
        in_specs=in_specs,
        out_specs=[
            pl.BlockSpec((bn, out_dim), lambda i: (i, 0)),
            pl.BlockSpec((bn, hid), lambda i: (i, 0)),
        ],
        out_shape=[
            jax.ShapeDtypeStruct((n, out_dim), jnp.float32),
            jax.ShapeDtypeStruct((n, hid), jnp.float32),
        ],
    )


def kernel(x, edge_index, W_mlp, b_mlp, W_filters, b_filters, W_cls, b_cls):
    n, d = x.shape
    hid = W_mlp.shape[1]
    out_dim = W_cls.shape[1]
    kp1 = W_filters.shape[0]
    e = edge_index.shape[1]

    e_pad = _cdiv(e, _NW * _EB) * _NW * _EB
    nbp = 2 * _cdiv(_cdiv(e, _NW * _EB), 2 * _NPH)
    e_pad2 = _NPH * nbp * _NW * _EB
    acc_rows = _cdiv(n + 1, _NS * 8) * _NS * 8
    rpt = acc_rows // _NS
    bn = 1000 if n % 1000 == 0 else (500 if n % 500 == 0 else 8)

    row = edge_index[0]
    col = edge_index[1]
    npad = e_pad - e
    if npad:
        row_d = jnp.concatenate([row, jnp.full((npad,), n, jnp.int32)])
    else:
        row_d = row
    npad2 = e_pad2 - e
    row4 = jnp.concatenate(
        [row, jnp.full((npad2,), n, jnp.int32)]).reshape(_NW, _NPH, nbp, _EB)
    col4 = jnp.concatenate(
        [col, jnp.zeros((npad2,), jnp.int32)]).reshape(_NW, _NPH, nbp, _EB)

    z_d = jnp.zeros((rpt, d), jnp.float32)
    z_16 = jnp.zeros((rpt, 16), jnp.float32)
    ones_16 = jnp.ones((_EB, 16), jnp.float32)

    sdeg = _build_deg(acc_rows, e_pad)(row_d, z_16, ones_16)

    h, hs, h0, qs0, dis, dinv = _build_mlp(n, d, hid, bn)(
        x, W_mlp, b_mlp.reshape(1, hid), W_filters[0],
        b_filters[0].reshape(1, hid), sdeg)

    a_c, b_c = 1.0, 1.0
    spmm = _build_spmm(n, d, acc_rows, nbp)
    rec = _build_rec(n, hid, bn)

    hs_list = [h0]
    qs_list = [qs0]
    zl, zp = h, h
    hs_cur = hs
    for k_idx in range(1, kp1):
        s = spmm(hs_cur, col4, row4, z_d)
        if k_idx == 1:
            c0 = (a_c + b_c + 2.0) / 2.0
            c1 = (a_c - b_c) / 2.0
            c2 = 0.0
        else:
            k = float(k_idx)
            c0 = (2 * k + a_c + b_c) * (2 * k + a_c + b_c - 1) / (
                2 * k * (k + a_c + b_c))
            c1 = (2 * k + a_c + b_c - 1) * (a_c ** 2 - b_c ** 2) / (
                2 * k * (k + a_c + b_c) * (2 * k + a_c + b_c - 2))
            c2 = -((k + a_c - 1) * (k + b_c - 1) * (2 * k + a_c + b_c) / (
                k * (k + a_c + b_c) * (2 * k + a_c + b_c - 2)))
        coefs = jnp.array([c0, c1, c2], jnp.float32)
        zn, hsn, hk, qsk = rec(coefs, s, zl, zp, dis, dinv,
                               W_filters[k_idx], b_filters[k_idx].reshape(1, hid))
        zp, zl, hs_cur = zl, zn, hsn
        hs_list.append(hk)
        qs_list.append(qsk)

    qsum = jnp.concatenate(qs_list, axis=0)
    out, zt = _build_attn(n, hid, out_dim, kp1, bn)(
        *hs_list, qsum, W_cls, b_cls.reshape(1, out_dim))
    return out, zt
